# Initial kernel scaffold; baseline (speedup 1.0000x reference)
import jax
import jax.numpy as jnp
from jax.experimental import pallas as pl


def kernel(x, edge_index, edge_weight, W1, b1, W2, b2, Wc, bc):
    raise NotImplementedError("write your pallas kernel here")


# speedup vs baseline: 8.4031x; 8.4031x over previous
import functools

import jax
import jax.numpy as jnp
from jax import lax
from jax.experimental import pallas as pl
from jax.experimental.pallas import tpu as pltpu
from jax.experimental.pallas import tpu_sc as plsc

NC = 2
NS = 16
NW = NC * NS
K = 128


def _ceil_to(a, m):
  return (a + m - 1) // m * m


def _make_deg_kernel(NPad, EPW):
  CH = EPW // K
  RPS = NPad // NS
  mesh = plsc.VectorSubcoreMesh(core_axis_name="c", subcore_axis_name="s",
                                num_cores=NC)

  @functools.partial(
      pl.kernel, mesh=mesh,
      out_type=jax.ShapeDtypeStruct((NC * NPad,), jnp.float32),
      scratch_types=[
          pltpu.VMEM((K,), jnp.int32),
          pltpu.VMEM((K,), jnp.float32),
          pltpu.VMEM_SHARED((NPad,), jnp.float32),
      ],
  )
  def deg_kernel(col_hbm, ew_hbm, out_hbm, cidx_v, ew_v, acc_sh):
    cid = lax.axis_index("c")
    sid = lax.axis_index("s")
    wid = sid * NC + cid

    for j in range(K // 16):
      ew_v[pl.ds(j * 16, 16)] = jnp.zeros((16,), jnp.float32)
    for t in range(RPS // K):
      pltpu.sync_copy(ew_v, acc_sh.at[pl.ds(sid * RPS + t * K, K)])
    plsc.subcore_barrier()

    base = wid * EPW

    def chunk(t, carry):
      e0 = base + t * K
      pltpu.sync_copy(col_hbm.at[pl.ds(e0, K)], cidx_v)
      pltpu.sync_copy(ew_hbm.at[pl.ds(e0, K)], ew_v)
      pltpu.sync_copy(ew_v, acc_sh.at[cidx_v], add=True)
      return carry

    lax.fori_loop(0, CH, chunk, 0)
    plsc.subcore_barrier()
    pltpu.sync_copy(acc_sh.at[pl.ds(sid * RPS, RPS)],
                    out_hbm.at[pl.ds(cid * NPad + sid * RPS, RPS)])

  return deg_kernel


def _make_msg_kernel(NPad, D, EPW):
  CH = EPW // K
  DV = D // 16
  RPS = NPad // NS
  mesh = plsc.VectorSubcoreMesh(core_axis_name="c", subcore_axis_name="s",
                                num_cores=NC)

  @functools.partial(
      pl.kernel, mesh=mesh,
      out_type=jax.ShapeDtypeStruct((NC * NPad, D), jnp.float32),
      scratch_types=[
          pltpu.VMEM((K,), jnp.int32),
          pltpu.VMEM((K,), jnp.int32),
          pltpu.VMEM((K,), jnp.float32),
          pltpu.VMEM((K, D), jnp.float32),
          pltpu.VMEM_SHARED((NPad, D), jnp.float32),
          pltpu.SemaphoreType.DMA,
      ],
  )
  def msg_kernel(h_hbm, row_hbm, col_hbm, ew_hbm, out_hbm,
                 ridx_v, cidx_v, ew_v, rows_v, acc_sh, sem):
    cid = lax.axis_index("c")
    sid = lax.axis_index("s")
    wid = sid * NC + cid

    def zero_row(e, carry):
      for j in range(DV):
        rows_v[e, pl.ds(j * 16, 16)] = jnp.zeros((16,), jnp.float32)
      return carry

    lax.fori_loop(0, K, zero_row, 0)
    for t in range(RPS // K):
      pltpu.sync_copy(rows_v, acc_sh.at[pl.ds(sid * RPS + t * K, K)])
    plsc.subcore_barrier()

    base = wid * EPW

    def chunk(t, carry):
      e0 = base + t * K
      pltpu.sync_copy(row_hbm.at[pl.ds(e0, K)], ridx_v)
      pltpu.sync_copy(col_hbm.at[pl.ds(e0, K)], cidx_v)
      pltpu.sync_copy(ew_hbm.at[pl.ds(e0, K)], ew_v)
      pltpu.async_copy(h_hbm.at[ridx_v], rows_v, sem).wait()

      def scale(g, c2):
        ew16 = ew_v[pl.ds(g * 16, 16)]
        for l in range(16):
          e = g * 16 + l
          s = ew16[l]
          for j in range(DV):
            rows_v[e, pl.ds(j * 16, 16)] = rows_v[e, pl.ds(j * 16, 16)] * s
        return c2

      lax.fori_loop(0, K // 16, scale, 0)
      pltpu.sync_copy(rows_v, acc_sh.at[cidx_v], add=True)
      return carry

    lax.fori_loop(0, CH, chunk, 0)
    plsc.subcore_barrier()
    pltpu.sync_copy(acc_sh.at[pl.ds(sid * RPS, RPS)],
                    out_hbm.at[pl.ds(cid * NPad + sid * RPS, RPS)])

  return msg_kernel


def _dinv_from(d_ref):
  deg = d_ref[:, 0:1] + d_ref[:, 1:2] + 1.0
  return jnp.where(deg > 0, lax.rsqrt(deg), 0.0)


def _dot(a, b):
  return lax.dot_general(a, b, (((1,), (0,)), ((), ())),
                         precision=lax.Precision.HIGHEST,
                         preferred_element_type=jnp.float32)


def _tc_stage1(xp, W1, degT, RB):
  NPad, D = xp.shape
  G = NPad // RB

  def body(x_ref, w_ref, d_ref, o_ref):
    o_ref[...] = _dot(x_ref[...], w_ref[...]) * _dinv_from(d_ref)

  return pl.pallas_call(
      body, grid=(G,),
      in_specs=[
          pl.BlockSpec((RB, D), lambda i: (i, 0)),
          pl.BlockSpec((D, W1.shape[1]), lambda i: (0, 0)),
          pl.BlockSpec((RB, 2), lambda i: (i, 0)),
      ],
      out_specs=pl.BlockSpec((RB, W1.shape[1]), lambda i: (i, 0)),
      out_shape=jax.ShapeDtypeStruct((NPad, W1.shape[1]), jnp.float32),
  )(xp, W1, degT)


def _tc_stage2(P1, H1p, degT, b1, W2, RB):
  NPad, D = H1p.shape
  G = NPad // RB

  def body(p_ref, h_ref, d_ref, b_ref, w_ref, o_ref):
    dinv = _dinv_from(d_ref)
    z = jnp.maximum(dinv * (p_ref[0] + p_ref[1] + h_ref[...]) + b_ref[...],
                    0.0)
    o_ref[...] = _dot(z, w_ref[...]) * dinv

  return pl.pallas_call(
      body, grid=(G,),
      in_specs=[
          pl.BlockSpec((2, RB, D), lambda i: (0, i, 0)),
          pl.BlockSpec((RB, D), lambda i: (i, 0)),
          pl.BlockSpec((RB, 2), lambda i: (i, 0)),
          pl.BlockSpec((1, D), lambda i: (0, 0)),
          pl.BlockSpec((D, W2.shape[1]), lambda i: (0, 0)),
      ],
      out_specs=pl.BlockSpec((RB, W2.shape[1]), lambda i: (i, 0)),
      out_shape=jax.ShapeDtypeStruct((NPad, W2.shape[1]), jnp.float32),
  )(P1, H1p, degT, b1, W2)


def _tc_stage3(P2, H2p, degT, b2, Wc, bc, RB):
  NPad, D = H2p.shape
  C = Wc.shape[1]
  G = NPad // RB

  def body(p_ref, h_ref, d_ref, b_ref, w_ref, bc_ref, o_ref):
    dinv = _dinv_from(d_ref)
    z = jnp.maximum(dinv * (p_ref[0] + p_ref[1] + h_ref[...]) + b_ref[...],
                    0.0)
    o_ref[...] = _dot(z, w_ref[...]) + bc_ref[...]

  return pl.pallas_call(
      body, grid=(G,),
      in_specs=[
          pl.BlockSpec((2, RB, D), lambda i: (0, i, 0)),
          pl.BlockSpec((RB, D), lambda i: (i, 0)),
          pl.BlockSpec((RB, 2), lambda i: (i, 0)),
          pl.BlockSpec((1, D), lambda i: (0, 0)),
          pl.BlockSpec((D, C), lambda i: (0, 0)),
          pl.BlockSpec((1, C), lambda i: (0, 0)),
      ],
      out_specs=pl.BlockSpec((RB, C), lambda i: (i, 0)),
      out_shape=jax.ShapeDtypeStruct((NPad, C), jnp.float32),
  )(P2, H2p, degT, b2, Wc, bc)


def kernel(x, edge_index, edge_weight, W1, b1, W2, b2, Wc, bc):
  N, D = x.shape
  E = edge_index.shape[1]

  EPW = _ceil_to(max(E // NW, 1), K)
  Epad = EPW * NW
  pe = Epad - E
  row = jnp.concatenate([edge_index[0], jnp.zeros((pe,), jnp.int32)])
  col = jnp.concatenate([edge_index[1], jnp.zeros((pe,), jnp.int32)])
  ew = jnp.concatenate([edge_weight, jnp.zeros((pe,), jnp.float32)])

  NPad = _ceil_to(N, NS * K)
  xp = jnp.concatenate([x, jnp.zeros((NPad - N, D), x.dtype)], axis=0)

  RB = 1024 if NPad % 1024 == 0 else K

  deg_parts = _make_deg_kernel(NPad, EPW)(col, ew)
  degT = jnp.transpose(deg_parts.reshape(NC, NPad))

  H1p = _tc_stage1(xp, W1, degT, RB)
  P1 = _make_msg_kernel(NPad, D, EPW)(H1p, row, col, ew)
  P1 = P1.reshape(NC, NPad, D)

  H2p = _tc_stage2(P1, H1p, degT, b1.reshape(1, -1), W2, RB)
  P2 = _make_msg_kernel(NPad, D, EPW)(H2p, row, col, ew)
  P2 = P2.reshape(NC, NPad, D)

  out = _tc_stage3(P2, H2p, degT, b2.reshape(1, -1), Wc,
                   bc.reshape(1, -1), RB)
  return out[:N]


# speedup vs baseline: 9.1835x; 1.0929x over previous
import functools

import jax
import jax.numpy as jnp
from jax import lax
from jax.experimental import pallas as pl
from jax.experimental.pallas import tpu as pltpu
from jax.experimental.pallas import tpu_sc as plsc

NC = 2
NS = 16
NW = NC * NS
K = 128


def _ceil_to(a, m):
  return (a + m - 1) // m * m


def _make_deg_kernel(NPad, EPW):
  CH = EPW // K
  RPS = NPad // NS
  mesh = plsc.VectorSubcoreMesh(core_axis_name="c", subcore_axis_name="s",
                                num_cores=NC)

  @functools.partial(
      pl.kernel, mesh=mesh,
      out_type=jax.ShapeDtypeStruct((NC * NPad,), jnp.float32),
      scratch_types=[
          pltpu.VMEM((K,), jnp.int32),
          pltpu.VMEM((K,), jnp.float32),
          pltpu.VMEM_SHARED((NPad,), jnp.float32),
      ],
  )
  def deg_kernel(col_hbm, ew_hbm, out_hbm, cidx_v, ew_v, acc_sh):
    cid = lax.axis_index("c")
    sid = lax.axis_index("s")
    wid = sid * NC + cid

    for j in range(K // 16):
      ew_v[pl.ds(j * 16, 16)] = jnp.zeros((16,), jnp.float32)
    for t in range(RPS // K):
      pltpu.sync_copy(ew_v, acc_sh.at[pl.ds(sid * RPS + t * K, K)])
    plsc.subcore_barrier()

    base = wid * EPW

    def chunk(t, carry):
      e0 = base + t * K
      pltpu.sync_copy(col_hbm.at[pl.ds(e0, K)], cidx_v)
      pltpu.sync_copy(ew_hbm.at[pl.ds(e0, K)], ew_v)
      pltpu.sync_copy(ew_v, acc_sh.at[cidx_v], add=True)
      return carry

    lax.fori_loop(0, CH, chunk, 0)
    plsc.subcore_barrier()
    pltpu.sync_copy(acc_sh.at[pl.ds(sid * RPS, RPS)],
                    out_hbm.at[pl.ds(cid * NPad + sid * RPS, RPS)])

  return deg_kernel


def _make_msg_kernel(NPad, D, EPW):
  CH = EPW // K
  G2 = CH // 2
  DV = D // 16
  RPS = NPad // NS
  mesh = plsc.VectorSubcoreMesh(core_axis_name="c", subcore_axis_name="s",
                                num_cores=NC)

  @functools.partial(
      pl.kernel, mesh=mesh,
      out_type=jax.ShapeDtypeStruct((NC * NPad, D), jnp.float32),
      scratch_types=[
          pltpu.VMEM((2, K), jnp.int32),
          pltpu.VMEM((2, K), jnp.int32),
          pltpu.VMEM((K,), jnp.float32),
          pltpu.VMEM((K,), jnp.float32),
          pltpu.VMEM((K, D), jnp.float32),
          pltpu.VMEM((K, D), jnp.float32),
          pltpu.VMEM_SHARED((NPad, D), jnp.float32),
          pltpu.SemaphoreType.DMA,
          pltpu.SemaphoreType.DMA,
      ],
  )
  def msg_kernel(h_hbm, ed_hbm, ewm_hbm, out_hbm,
                 ed_a, ed_b, ew_a, ew_b, rows_a, rows_b, acc_sh,
                 sem_a, sem_b):
    cid = lax.axis_index("c")
    sid = lax.axis_index("s")
    wid = sid * NC + cid

    def scale_rows(ew_v, rows_v):
      def scale(g, c2):
        w16 = ew_v[pl.ds(g * 16, 16)]
        for l in range(16):
          e = g * 16 + l
          s = w16[l]
          for j in range(DV):
            rows_v[e, pl.ds(j * 16, 16)] = rows_v[e, pl.ds(j * 16, 16)] * s
        return c2

      lax.fori_loop(0, K // 16, scale, 0)

    def zero_row(e, carry):
      for j in range(DV):
        rows_a[e, pl.ds(j * 16, 16)] = jnp.zeros((16,), jnp.float32)
      return carry

    lax.fori_loop(0, K, zero_row, 0)
    for t in range(RPS // K):
      pltpu.sync_copy(rows_a, acc_sh.at[pl.ds(sid * RPS + t * K, K)])
    plsc.subcore_barrier()

    cbase = wid * CH

    pltpu.sync_copy(ed_hbm.at[cbase], ed_a)
    pltpu.sync_copy(ewm_hbm.at[cbase], ew_a)
    pltpu.async_copy(h_hbm.at[ed_a.at[0]], rows_a, sem_a)

    def pair(g, carry):
      c = cbase + 2 * g
      pltpu.sync_copy(ed_hbm.at[c + 1], ed_b)
      pltpu.sync_copy(ewm_hbm.at[c + 1], ew_b)
      pltpu.async_copy(h_hbm.at[ed_b.at[0]], rows_b, sem_b)
      pltpu.make_async_copy(h_hbm.at[ed_a.at[0]], rows_a, sem_a).wait()
      scale_rows(ew_a, rows_a)
      pltpu.sync_copy(rows_a, acc_sh.at[ed_a.at[1]], add=True)

      @pl.when(g + 1 < G2)
      def _():
        pltpu.sync_copy(ed_hbm.at[c + 2], ed_a)
        pltpu.sync_copy(ewm_hbm.at[c + 2], ew_a)
        pltpu.async_copy(h_hbm.at[ed_a.at[0]], rows_a, sem_a)

      pltpu.make_async_copy(h_hbm.at[ed_b.at[0]], rows_b, sem_b).wait()
      scale_rows(ew_b, rows_b)
      pltpu.sync_copy(rows_b, acc_sh.at[ed_b.at[1]], add=True)
      return carry

    lax.fori_loop(0, G2, pair, 0)
    plsc.subcore_barrier()
    pltpu.sync_copy(acc_sh.at[pl.ds(sid * RPS, RPS)],
                    out_hbm.at[pl.ds(cid * NPad + sid * RPS, RPS)])

  return msg_kernel


def _dinv_from(d_ref):
  deg = d_ref[:, 0:1] + d_ref[:, 1:2] + 1.0
  return jnp.where(deg > 0, lax.rsqrt(deg), 0.0)


def _dot(a, b):
  return lax.dot_general(a, b, (((1,), (0,)), ((), ())),
                         precision=lax.Precision.HIGHEST,
                         preferred_element_type=jnp.float32)


def _tc_stage1(xp, W1, degT, RB):
  NPad, D = xp.shape
  G = NPad // RB

  def body(x_ref, w_ref, d_ref, o_ref):
    o_ref[...] = _dot(x_ref[...], w_ref[...]) * _dinv_from(d_ref)

  return pl.pallas_call(
      body, grid=(G,),
      in_specs=[
          pl.BlockSpec((RB, D), lambda i: (i, 0)),
          pl.BlockSpec((D, W1.shape[1]), lambda i: (0, 0)),
          pl.BlockSpec((RB, 2), lambda i: (i, 0)),
      ],
      out_specs=pl.BlockSpec((RB, W1.shape[1]), lambda i: (i, 0)),
      out_shape=jax.ShapeDtypeStruct((NPad, W1.shape[1]), jnp.float32),
  )(xp, W1, degT)


def _tc_stage2(P1, H1p, degT, b1, W2, RB):
  NPad, D = H1p.shape
  G = NPad // RB

  def body(p_ref, h_ref, d_ref, b_ref, w_ref, o_ref):
    dinv = _dinv_from(d_ref)
    z = jnp.maximum(dinv * (p_ref[0] + p_ref[1] + h_ref[...]) + b_ref[...],
                    0.0)
    o_ref[...] = _dot(z, w_ref[...]) * dinv

  return pl.pallas_call(
      body, grid=(G,),
      in_specs=[
          pl.BlockSpec((2, RB, D), lambda i: (0, i, 0)),
          pl.BlockSpec((RB, D), lambda i: (i, 0)),
          pl.BlockSpec((RB, 2), lambda i: (i, 0)),
          pl.BlockSpec((1, D), lambda i: (0, 0)),
          pl.BlockSpec((D, W2.shape[1]), lambda i: (0, 0)),
      ],
      out_specs=pl.BlockSpec((RB, W2.shape[1]), lambda i: (i, 0)),
      out_shape=jax.ShapeDtypeStruct((NPad, W2.shape[1]), jnp.float32),
  )(P1, H1p, degT, b1, W2)


def _tc_stage3(P2, H2p, degT, b2, Wc, bc, RB):
  NPad, D = H2p.shape
  C = Wc.shape[1]
  G = NPad // RB

  def body(p_ref, h_ref, d_ref, b_ref, w_ref, bc_ref, o_ref):
    dinv = _dinv_from(d_ref)
    z = jnp.maximum(dinv * (p_ref[0] + p_ref[1] + h_ref[...]) + b_ref[...],
                    0.0)
    o_ref[...] = _dot(z, w_ref[...]) + bc_ref[...]

  return pl.pallas_call(
      body, grid=(G,),
      in_specs=[
          pl.BlockSpec((2, RB, D), lambda i: (0, i, 0)),
          pl.BlockSpec((RB, D), lambda i: (i, 0)),
          pl.BlockSpec((RB, 2), lambda i: (i, 0)),
          pl.BlockSpec((1, D), lambda i: (0, 0)),
          pl.BlockSpec((D, C), lambda i: (0, 0)),
          pl.BlockSpec((1, C), lambda i: (0, 0)),
      ],
      out_specs=pl.BlockSpec((RB, C), lambda i: (i, 0)),
      out_shape=jax.ShapeDtypeStruct((NPad, C), jnp.float32),
  )(P2, H2p, degT, b2, Wc, bc)


def kernel(x, edge_index, edge_weight, W1, b1, W2, b2, Wc, bc):
  N, D = x.shape
  E = edge_index.shape[1]

  EPW = _ceil_to(max(E // NW, 1), 2 * K)
  Epad = EPW * NW
  pe = Epad - E
  row = jnp.concatenate([edge_index[0], jnp.zeros((pe,), jnp.int32)])
  col = jnp.concatenate([edge_index[1], jnp.zeros((pe,), jnp.int32)])
  ew = jnp.concatenate([edge_weight, jnp.zeros((pe,), jnp.float32)])
  CHT = Epad // K
  ed = jnp.stack([row.reshape(CHT, K), col.reshape(CHT, K)], axis=1)
  ewm = ew.reshape(CHT, K)

  NPad = _ceil_to(N, NS * K)
  xp = jnp.concatenate([x, jnp.zeros((NPad - N, D), x.dtype)], axis=0)

  RB = 1024 if NPad % 1024 == 0 else K

  deg_parts = _make_deg_kernel(NPad, EPW)(col, ew)
  degT = jnp.transpose(deg_parts.reshape(NC, NPad))

  H1p = _tc_stage1(xp, W1, degT, RB)
  P1 = _make_msg_kernel(NPad, D, EPW)(H1p, ed, ewm)
  P1 = P1.reshape(NC, NPad, D)

  H2p = _tc_stage2(P1, H1p, degT, b1.reshape(1, -1), W2, RB)
  P2 = _make_msg_kernel(NPad, D, EPW)(H2p, ed, ewm)
  P2 = P2.reshape(NC, NPad, D)

  out = _tc_stage3(P2, H2p, degT, b2.reshape(1, -1), Wc,
                   bc.reshape(1, -1), RB)
  return out[:N]


# speedup vs baseline: 9.5375x; 1.0385x over previous
import functools
import math

import jax
import jax.numpy as jnp
from jax import lax
from jax.experimental import pallas as pl
from jax.experimental.pallas import tpu as pltpu
from jax.experimental.pallas import tpu_sc as plsc

NC = 2
NS = 16
NW = NC * NS
K = 128


def _ceil_to(a, m):
  return (a + m - 1) // m * m


def _make_deg_kernel(NPad, EPW):
  CH = EPW // K
  RPS = NPad // NS
  mesh = plsc.VectorSubcoreMesh(core_axis_name="c", subcore_axis_name="s",
                                num_cores=NC)

  @functools.partial(
      pl.kernel, mesh=mesh,
      out_type=jax.ShapeDtypeStruct((NC * NPad,), jnp.float32),
      scratch_types=[
          pltpu.VMEM((K,), jnp.int32),
          pltpu.VMEM((K,), jnp.float32),
          pltpu.VMEM_SHARED((NPad,), jnp.float32),
      ],
  )
  def deg_kernel(col_hbm, ew_hbm, out_hbm, cidx_v, ew_v, acc_sh):
    cid = lax.axis_index("c")
    sid = lax.axis_index("s")
    wid = sid * NC + cid

    for j in range(K // 16):
      ew_v[pl.ds(j * 16, 16)] = jnp.zeros((16,), jnp.float32)
    for t in range(RPS // K):
      pltpu.sync_copy(ew_v, acc_sh.at[pl.ds(sid * RPS + t * K, K)])
    plsc.subcore_barrier()

    base = wid * EPW

    def chunk(t, carry):
      e0 = base + t * K
      pltpu.sync_copy(col_hbm.at[pl.ds(e0, K)], cidx_v)
      pltpu.sync_copy(ew_hbm.at[pl.ds(e0, K)], ew_v)
      pltpu.sync_copy(ew_v, acc_sh.at[cidx_v], add=True)
      return carry

    lax.fori_loop(0, CH, chunk, 0)
    plsc.subcore_barrier()
    pltpu.sync_copy(acc_sh.at[pl.ds(sid * RPS, RPS)],
                    out_hbm.at[pl.ds(cid * NPad + sid * RPS, RPS)])

  return deg_kernel


NSET = 4
KM = 80


def _make_msg_kernel(NPad, D, EPW):
  CH = EPW // KM
  GG = CH // NSET
  DV = D // 16
  RPS = NPad // NS
  mesh = plsc.VectorSubcoreMesh(core_axis_name="c", subcore_axis_name="s",
                                num_cores=NC)

  scratch = ([pltpu.VMEM((2, KM), jnp.int32)] * NSET +
             [pltpu.VMEM((KM,), jnp.float32)] * NSET +
             [pltpu.VMEM((KM, D), jnp.float32)] * NSET +
             [pltpu.VMEM_SHARED((NPad, D), jnp.float32)] +
             [pltpu.SemaphoreType.DMA] * (2 * NSET))

  @functools.partial(
      pl.kernel, mesh=mesh,
      out_type=jax.ShapeDtypeStruct((NC * NPad, D), jnp.float32),
      scratch_types=scratch,
  )
  def msg_kernel(h_hbm, ed_hbm, ewm_hbm, out_hbm, *bufs):
    ed = bufs[0:NSET]
    ewv = bufs[NSET:2 * NSET]
    rows = bufs[2 * NSET:3 * NSET]
    acc_sh = bufs[3 * NSET]
    gs = bufs[3 * NSET + 1:4 * NSET + 1]
    ss = bufs[4 * NSET + 1:5 * NSET + 1]
    cid = lax.axis_index("c")
    sid = lax.axis_index("s")
    wid = sid * NC + cid

    def scale_rows(ew_v, rows_v):
      def scale(g, c2):
        w16 = ew_v[pl.ds(g * 16, 16)]
        for l in range(16):
          e = g * 16 + l
          s = w16[l]
          for j in range(DV):
            rows_v[e, pl.ds(j * 16, 16)] = rows_v[e, pl.ds(j * 16, 16)] * s
        return c2

      lax.fori_loop(0, KM // 16, scale, 0)

    def zero_row(e, carry):
      for j in range(DV):
        rows[0][e, pl.ds(j * 16, 16)] = jnp.zeros((16,), jnp.float32)
      return carry

    lax.fori_loop(0, KM, zero_row, 0)
    for t in range(RPS // KM):
      pltpu.sync_copy(rows[0], acc_sh.at[pl.ds(sid * RPS + t * KM, KM)])
    plsc.subcore_barrier()

    cbase = wid * CH

    def fetch(s, c):
      pltpu.sync_copy(ed_hbm.at[c], ed[s])
      pltpu.sync_copy(ewm_hbm.at[c], ewv[s])
      pltpu.async_copy(h_hbm.at[ed[s].at[0]], rows[s], gs[s])

    def wait_scatter(s):
      pltpu.make_async_copy(rows[s], acc_sh.at[ed[s].at[1]], ss[s]).wait()

    fetch(0, cbase)
    fetch(1, cbase + 1)

    def group(g, carry):
      c0 = NSET * g
      for s in range(NSET):
        c = c0 + s
        pltpu.make_async_copy(h_hbm.at[ed[s].at[0]], rows[s], gs[s]).wait()
        scale_rows(ewv[s], rows[s])
        pltpu.async_copy(rows[s], acc_sh.at[ed[s].at[1]], ss[s], add=True)
        pf = (s + 2) % NSET

        @pl.when(c + 2 < CH)
        def _():
          @pl.when(c >= 2)
          def _():
            wait_scatter(pf)

          fetch(pf, cbase + c + 2)

      return carry

    lax.fori_loop(0, GG, group, 0)
    for s in range(NSET):
      wait_scatter(s)
    plsc.subcore_barrier()
    pltpu.sync_copy(acc_sh.at[pl.ds(sid * RPS, RPS)],
                    out_hbm.at[pl.ds(cid * NPad + sid * RPS, RPS)])

  return msg_kernel


def _dinv_from(d_ref):
  deg = d_ref[:, 0:1] + d_ref[:, 1:2] + 1.0
  return jnp.where(deg > 0, lax.rsqrt(deg), 0.0)


def _dot(a, b):
  return lax.dot_general(a, b, (((1,), (0,)), ((), ())),
                         precision=lax.Precision.HIGHEST,
                         preferred_element_type=jnp.float32)


def _tc_stage1(xp, W1, degT, RB):
  NPad, D = xp.shape
  G = NPad // RB

  def body(x_ref, w_ref, d_ref, o_ref):
    o_ref[...] = _dot(x_ref[...], w_ref[...]) * _dinv_from(d_ref)

  return pl.pallas_call(
      body, grid=(G,),
      in_specs=[
          pl.BlockSpec((RB, D), lambda i: (i, 0)),
          pl.BlockSpec((D, W1.shape[1]), lambda i: (0, 0)),
          pl.BlockSpec((RB, 2), lambda i: (i, 0)),
      ],
      out_specs=pl.BlockSpec((RB, W1.shape[1]), lambda i: (i, 0)),
      out_shape=jax.ShapeDtypeStruct((NPad, W1.shape[1]), jnp.float32),
  )(xp, W1, degT)


def _tc_stage2(P1, H1p, degT, b1, W2, RB):
  NPad, D = H1p.shape
  G = NPad // RB

  def body(p_ref, h_ref, d_ref, b_ref, w_ref, o_ref):
    dinv = _dinv_from(d_ref)
    z = jnp.maximum(dinv * (p_ref[0] + p_ref[1] + h_ref[...]) + b_ref[...],
                    0.0)
    o_ref[...] = _dot(z, w_ref[...]) * dinv

  return pl.pallas_call(
      body, grid=(G,),
      in_specs=[
          pl.BlockSpec((2, RB, D), lambda i: (0, i, 0)),
          pl.BlockSpec((RB, D), lambda i: (i, 0)),
          pl.BlockSpec((RB, 2), lambda i: (i, 0)),
          pl.BlockSpec((1, D), lambda i: (0, 0)),
          pl.BlockSpec((D, W2.shape[1]), lambda i: (0, 0)),
      ],
      out_specs=pl.BlockSpec((RB, W2.shape[1]), lambda i: (i, 0)),
      out_shape=jax.ShapeDtypeStruct((NPad, W2.shape[1]), jnp.float32),
  )(P1, H1p, degT, b1, W2)


def _tc_stage3(P2, H2p, degT, b2, Wc, bc, RB):
  NPad, D = H2p.shape
  C = Wc.shape[1]
  G = NPad // RB

  def body(p_ref, h_ref, d_ref, b_ref, w_ref, bc_ref, o_ref):
    dinv = _dinv_from(d_ref)
    z = jnp.maximum(dinv * (p_ref[0] + p_ref[1] + h_ref[...]) + b_ref[...],
                    0.0)
    o_ref[...] = _dot(z, w_ref[...]) + bc_ref[...]

  return pl.pallas_call(
      body, grid=(G,),
      in_specs=[
          pl.BlockSpec((2, RB, D), lambda i: (0, i, 0)),
          pl.BlockSpec((RB, D), lambda i: (i, 0)),
          pl.BlockSpec((RB, 2), lambda i: (i, 0)),
          pl.BlockSpec((1, D), lambda i: (0, 0)),
          pl.BlockSpec((D, C), lambda i: (0, 0)),
          pl.BlockSpec((1, C), lambda i: (0, 0)),
      ],
      out_specs=pl.BlockSpec((RB, C), lambda i: (i, 0)),
      out_shape=jax.ShapeDtypeStruct((NPad, C), jnp.float32),
  )(P2, H2p, degT, b2, Wc, bc)


def kernel(x, edge_index, edge_weight, W1, b1, W2, b2, Wc, bc):
  N, D = x.shape
  E = edge_index.shape[1]

  EPW = _ceil_to(max(E // NW, 1), math.lcm(K, NSET * KM))
  Epad = EPW * NW
  pe = Epad - E
  row = jnp.concatenate([edge_index[0], jnp.zeros((pe,), jnp.int32)])
  col = jnp.concatenate([edge_index[1], jnp.zeros((pe,), jnp.int32)])
  ew = jnp.concatenate([edge_weight, jnp.zeros((pe,), jnp.float32)])
  CHT = Epad // KM
  ed = jnp.stack([row.reshape(CHT, KM), col.reshape(CHT, KM)], axis=1)
  ewm = ew.reshape(CHT, KM)

  NPad = _ceil_to(N, NS * math.lcm(K, KM))
  xp = jnp.concatenate([x, jnp.zeros((NPad - N, D), x.dtype)], axis=0)

  RB = 1024 if NPad % 1024 == 0 else K

  deg_parts = _make_deg_kernel(NPad, EPW)(col, ew)
  degT = jnp.transpose(deg_parts.reshape(NC, NPad))

  H1p = _tc_stage1(xp, W1, degT, RB)
  P1 = _make_msg_kernel(NPad, D, EPW)(H1p, ed, ewm)
  P1 = P1.reshape(NC, NPad, D)

  H2p = _tc_stage2(P1, H1p, degT, b1.reshape(1, -1), W2, RB)
  P2 = _make_msg_kernel(NPad, D, EPW)(H2p, ed, ewm)
  P2 = P2.reshape(NC, NPad, D)

  out = _tc_stage3(P2, H2p, degT, b2.reshape(1, -1), Wc,
                   bc.reshape(1, -1), RB)
  return out[:N]
